# Initial kernel scaffold; baseline (speedup 1.0000x reference)
import jax
import jax.numpy as jnp
from jax.experimental import pallas as pl


def kernel(x, weight):
    raise NotImplementedError("write your pallas kernel here")


# speedup vs baseline: 3.0600x; 3.0600x over previous
import functools

import jax
import jax.numpy as jnp
from jax import lax
from jax.experimental import pallas as pl
from jax.experimental.pallas import tpu as pltpu
from jax.experimental.pallas import tpu_sc as plsc

D = 128
N = 16384 * 50
NC, NS = 2, 16
NW = NC * NS
PER_W = N // NW
CH = 128
NCH = PER_W // CH
NBUF = 2


def _emb_body(x_hbm, w_hbm, out_hbm, idx_v, rows_v, sem_in):
    wid = lax.axis_index("s") * NC + lax.axis_index("c")
    cbase = wid * NCH

    pltpu.sync_copy(x_hbm.at[pl.ds(cbase, NCH)], idx_v)

    @pl.loop(0, NCH)
    def chunk_loop(j):
        pltpu.async_copy(w_hbm.at[idx_v.at[j]], rows_v, sem_in).wait()
        pltpu.sync_copy(rows_v, out_hbm.at[cbase + j])


@jax.jit
def _emb_lookup(xf, weight):
    mesh = plsc.VectorSubcoreMesh(core_axis_name="c", subcore_axis_name="s")
    run = pl.kernel(
        _emb_body,
        out_type=jax.ShapeDtypeStruct((NW * NCH, CH, D), jnp.float32),
        mesh=mesh,
        scratch_types=[
            pltpu.VMEM((NCH, CH), jnp.int32),
            pltpu.VMEM((CH, D), jnp.float32),
            pltpu.SemaphoreType.DMA,
        ],
    )
    return run(xf, weight)


def kernel(x, weight):
    B, L = x.shape
    xf = x.reshape(NW * NCH, CH).astype(jnp.int32)
    out = _emb_lookup(xf, weight)
    return out.reshape(B, L, D)


# speedup vs baseline: 3.2439x; 1.0601x over previous
import functools

import jax
import jax.numpy as jnp
from jax import lax
from jax.experimental import pallas as pl
from jax.experimental.pallas import tpu as pltpu
from jax.experimental.pallas import tpu_sc as plsc

D = 128
N = 16384 * 50
NC, NS = 2, 16
NW = NC * NS
PER_W = N // NW
CH = 128
NCH = PER_W // CH
NBUF = 2


NG = NCH


def _emb_body(x_hbm, w_hbm, out_hbm, idx_v, rows_v, sem_g, sem_w0, sem_w1):
    wid = lax.axis_index("s") * NC + lax.axis_index("c")
    gbase = wid * NG
    sem_w = (sem_w0, sem_w1)

    pltpu.sync_copy(x_hbm.at[pl.ds(gbase, NG)], idx_v)

    def gather(g, buf):
        pltpu.async_copy(w_hbm.at[idx_v.at[g - gbase]], rows_v.at[buf], sem_g).wait()

    def fire_write(g, buf):
        pltpu.async_copy(rows_v.at[buf], out_hbm.at[g], sem_w[buf])

    def wait_write(g, buf):
        pltpu.make_async_copy(rows_v.at[buf], out_hbm.at[g], sem_w[buf]).wait()

    gather(gbase, 0)
    fire_write(gbase, 0)

    @pl.loop(gbase + 1, gbase + NG - 1, step=2)
    def group_loop(g):
        gather(g, 1)
        fire_write(g, 1)
        wait_write(g - 1, 0)
        gather(g + 1, 0)
        fire_write(g + 1, 0)
        wait_write(g, 1)

    g_last = gbase + NG - 1
    gather(g_last, 1)
    fire_write(g_last, 1)
    wait_write(g_last - 1, 0)
    wait_write(g_last, 1)


@jax.jit
def _emb_lookup(xf, weight):
    mesh = plsc.VectorSubcoreMesh(core_axis_name="c", subcore_axis_name="s")
    run = pl.kernel(
        _emb_body,
        out_type=jax.ShapeDtypeStruct((NW * NG, CH, D), jnp.float32),
        mesh=mesh,
        scratch_types=[
            pltpu.VMEM((NG, CH), jnp.int32),
            pltpu.VMEM((2, CH, D), jnp.float32),
            pltpu.SemaphoreType.DMA,
            pltpu.SemaphoreType.DMA,
            pltpu.SemaphoreType.DMA,
        ],
    )
    return run(xf, weight)


def kernel(x, weight):
    B, L = x.shape
    xf = x.reshape(NW * NG, CH).astype(jnp.int32)
    out = _emb_lookup(xf, weight)
    return out.reshape(B, L, D)


# speedup vs baseline: 3.4662x; 1.0685x over previous
import functools

import jax
import jax.numpy as jnp
from jax import lax
from jax.experimental import pallas as pl
from jax.experimental.pallas import tpu as pltpu
from jax.experimental.pallas import tpu_sc as plsc

D = 128
N = 16384 * 50
NC, NS = 2, 16
NW = NC * NS
PER_W = N // NW
CH = 128
NCH = PER_W // CH
NBUF = 2


NG = NCH
NBUF = 4
A = 2


def _emb_body(x_hbm, w_hbm, out_hbm, idx_v, rows_v,
              sg0, sg1, sg2, sg3, sw0, sw1, sw2, sw3):
    semg = (sg0, sg1, sg2, sg3)
    semw = (sw0, sw1, sw2, sw3)
    wid = lax.axis_index("s") * NC + lax.axis_index("c")
    gbase = wid * NG

    pltpu.sync_copy(x_hbm.at[pl.ds(gbase, NG)], idx_v)

    def fire_g(j, b):
        pltpu.async_copy(w_hbm.at[idx_v.at[j]], rows_v.at[b], semg[b])

    def wait_g(j, b):
        pltpu.make_async_copy(w_hbm.at[idx_v.at[j]], rows_v.at[b], semg[b]).wait()

    def fire_w(j, b):
        pltpu.async_copy(rows_v.at[b], out_hbm.at[gbase + j], semw[b])

    def wait_w(j, b):
        pltpu.make_async_copy(rows_v.at[b], out_hbm.at[gbase + j], semw[b]).wait()

    fire_g(0, 0)
    fire_g(1, 1)
    wait_g(0, 0); fire_w(0, 0); fire_g(2, 2)
    wait_g(1, 1); fire_w(1, 1); fire_g(3, 3)

    @pl.loop(2, NG - 2, step=NBUF)
    def visit_loop(j0):
        for k in range(NBUF):
            j = j0 + k
            b = (2 + k) % NBUF
            bn = (b + A) % NBUF
            wait_g(j, b)
            fire_w(j, b)
            wait_w(j - A, bn)
            fire_g(j + A, bn)

    wait_g(NG - 2, 2); fire_w(NG - 2, 2); wait_w(NG - 4, 0)
    wait_g(NG - 1, 3); fire_w(NG - 1, 3); wait_w(NG - 3, 1)
    wait_w(NG - 2, 2)
    wait_w(NG - 1, 3)


@jax.jit
def _emb_lookup(xf, weight):
    mesh = plsc.VectorSubcoreMesh(core_axis_name="c", subcore_axis_name="s")
    run = pl.kernel(
        _emb_body,
        out_type=jax.ShapeDtypeStruct((NW * NG, CH, D), jnp.float32),
        mesh=mesh,
        scratch_types=[
            pltpu.VMEM((NG, CH), jnp.int32),
            pltpu.VMEM((NBUF, CH, D), jnp.float32),
        ] + [pltpu.SemaphoreType.DMA] * (2 * NBUF),
    )
    return run(xf, weight)


def kernel(x, weight):
    B, L = x.shape
    xf = x.reshape(NW * NG, CH).astype(jnp.int32)
    out = _emb_lookup(xf, weight)
    return out.reshape(B, L, D)


# speedup vs baseline: 6.3323x; 1.8269x over previous
import functools

import jax
import jax.numpy as jnp
from jax import lax
from jax.experimental import pallas as pl
from jax.experimental.pallas import tpu as pltpu
from jax.experimental.pallas import tpu_sc as plsc

D = 128
B, L = 16384, 50
NC, NS = 2, 16
NW = NC * NS
PB = 2
RPC = PB * L
NG = B // (PB * NW)
NBUF = 4
A = 2


def _emb_body(x_hbm, w_hbm, out_hbm, idx_v, rows_v,
              sg0, sg1, sg2, sg3, sw0, sw1, sw2, sw3):
    semg = (sg0, sg1, sg2, sg3)
    semw = (sw0, sw1, sw2, sw3)
    wid = lax.axis_index("s") * NC + lax.axis_index("c")
    gbase = wid * NG

    pltpu.sync_copy(x_hbm.at[pl.ds(gbase, NG)], idx_v)

    def fire_g(j, b):
        pltpu.async_copy(
            w_hbm.at[idx_v.at[j, pl.ds(0, RPC)]], rows_v.at[b], semg[b])

    def wait_g(j, b):
        pltpu.make_async_copy(
            w_hbm.at[idx_v.at[j, pl.ds(0, RPC)]], rows_v.at[b], semg[b]
        ).wait()

    def fire_w(j, b):
        p = (gbase + j) * PB
        pltpu.async_copy(rows_v.at[b, pl.ds(0, L)], out_hbm.at[p], semw[b])
        pltpu.async_copy(rows_v.at[b, pl.ds(L, L)], out_hbm.at[p + 1], semw[b])

    def wait_w(j, b):
        p = (gbase + j) * PB
        pltpu.make_async_copy(
            rows_v.at[b, pl.ds(0, L)], out_hbm.at[p], semw[b]).wait()
        pltpu.make_async_copy(
            rows_v.at[b, pl.ds(L, L)], out_hbm.at[p + 1], semw[b]).wait()

    fire_g(0, 0)
    fire_g(1, 1)
    wait_g(0, 0); fire_w(0, 0); fire_g(2, 2)
    wait_g(1, 1); fire_w(1, 1); fire_g(3, 3)

    @pl.loop(2, NG - 2, step=NBUF)
    def visit_loop(j0):
        for k in range(NBUF):
            j = j0 + k
            b = (2 + k) % NBUF
            bn = (b + A) % NBUF
            wait_g(j, b)
            fire_w(j, b)
            wait_w(j - A, bn)
            fire_g(j + A, bn)

    wait_g(NG - 2, 2); fire_w(NG - 2, 2); wait_w(NG - 4, 0)
    wait_g(NG - 1, 3); fire_w(NG - 1, 3); wait_w(NG - 3, 1)
    wait_w(NG - 2, 2)
    wait_w(NG - 1, 3)


@jax.jit
def _emb_lookup(xf, weight):
    mesh = plsc.VectorSubcoreMesh(core_axis_name="c", subcore_axis_name="s")
    run = pl.kernel(
        _emb_body,
        out_type=jax.ShapeDtypeStruct((B, L, D), jnp.float32),
        mesh=mesh,
        scratch_types=[
            pltpu.VMEM((NG, 128), jnp.int32),
            pltpu.VMEM((NBUF, RPC, D), jnp.float32),
        ] + [pltpu.SemaphoreType.DMA] * (2 * NBUF),
    )
    return run(xf, weight)


def kernel(x, weight):
    xf = jnp.pad(x.reshape(B // PB, PB * L).astype(jnp.int32),
                 ((0, 0), (0, 128 - RPC)))
    return _emb_lookup(xf, weight)
